# baseline (speedup 1.0000x reference)
import functools

import jax
import jax.numpy as jnp
from jax import lax
from jax.experimental import pallas as pl
from jax.experimental.pallas import tpu as pltpu
from jax.experimental.pallas import tpu_sc as plsc

B = 4096
L = 200
D = 64
NW = 32
BPW = B // NW
LG = D // 16
LH = L // 2


def _sc_body(text_hbm, len_hbm, table_hbm, embs_hbm, seq_hbm,
             idx_all, rows_v, seq_acc, lens_v, gsem):
    c = lax.axis_index("c")
    s = lax.axis_index("s")
    wid = s * 2 + c
    base = wid * BPW

    pltpu.sync_copy(text_hbm.at[pl.ds(base, BPW)], idx_all)
    pltpu.sync_copy(len_hbm.at[pl.ds(base, BPW)], lens_v.at[pl.ds(0, BPW)])

    def row(r, carry):
        cp0 = pltpu.async_copy(table_hbm.at[idx_all.at[r, 0]],
                               rows_v.at[pl.ds(0, LH)], gsem)
        cp1 = pltpu.async_copy(table_hbm.at[idx_all.at[r, 1]],
                               rows_v.at[pl.ds(LH, LH)], gsem)
        cp0.wait()
        cp1.wait()

        pltpu.sync_copy(rows_v, embs_hbm.at[pl.ds((base + r) * L, L)])

        ln = lens_v[pl.ds(r, 16)][0]

        def acc_body(i, acc):
            return tuple(acc[g] + rows_v[i, pl.ds(g * 16, 16)]
                         for g in range(LG))

        acc = lax.fori_loop(
            0, ln, acc_body,
            tuple(jnp.zeros((16,), jnp.float32) for _ in range(LG)))
        lf_v = jnp.full((16,), ln.astype(jnp.float32))
        for g in range(LG):
            seq_acc[r, pl.ds(g * 16, 16)] = acc[g] / lf_v
        return carry

    lax.fori_loop(0, BPW, row, 0)
    pltpu.sync_copy(seq_acc, seq_hbm.at[pl.ds(base, BPW)])


@functools.partial(jax.jit, static_argnames=())
def _run(text_r, text_len, table):
    mesh = plsc.VectorSubcoreMesh(core_axis_name="c", subcore_axis_name="s")
    k = pl.kernel(
        _sc_body,
        mesh=mesh,
        out_type=[
            jax.ShapeDtypeStruct((B * L, D), jnp.float32),
            jax.ShapeDtypeStruct((B, D), jnp.float32),
        ],
        scratch_types=[
            pltpu.VMEM((BPW, 2, LH), jnp.int32),
            pltpu.VMEM((L, D), jnp.float32),
            pltpu.VMEM((BPW, D), jnp.float32),
            pltpu.VMEM((BPW + 16,), jnp.int32),
            pltpu.SemaphoreType.DMA,
        ],
        compiler_params=pltpu.CompilerParams(use_tc_tiling_on_sc=False),
    )
    return k(text_r, text_len, table)


def kernel(text, text_len, table):
    text_r = text.astype(jnp.int32).reshape(B, 2, LH)
    embs_flat, seq = _run(text_r, text_len.astype(jnp.int32), table)
    return embs_flat.reshape(B, L, D), seq


# speedup vs baseline: 1.1116x; 1.1116x over previous
import functools

import jax
import jax.numpy as jnp
from jax import lax
from jax.experimental import pallas as pl
from jax.experimental.pallas import tpu as pltpu
from jax.experimental.pallas import tpu_sc as plsc

B = 4096
L = 200
D = 64
NW = 32
BPW = B // NW
LG = D // 16
LH = L // 2


NBUF = 4


def _sc_body(text_hbm, len_hbm, table_hbm, embs_hbm, seq_hbm,
             idx_all, seq_acc, lens_v, *bufs_and_sems):
    rows_bufs = bufs_and_sems[:NBUF]
    gsems = bufs_and_sems[NBUF:2 * NBUF]
    osems = bufs_and_sems[2 * NBUF:3 * NBUF]
    c = lax.axis_index("c")
    s = lax.axis_index("s")
    wid = s * 2 + c
    base = wid * BPW

    pltpu.sync_copy(text_hbm.at[pl.ds(base, BPW)], idx_all)
    pltpu.sync_copy(len_hbm.at[pl.ds(base, BPW)], lens_v.at[pl.ds(0, BPW)])

    def start_gather(r, b):
        pltpu.async_copy(table_hbm.at[idx_all.at[r, 0]],
                         rows_bufs[b].at[pl.ds(0, LH)], gsems[b])
        pltpu.async_copy(table_hbm.at[idx_all.at[r, 1]],
                         rows_bufs[b].at[pl.ds(LH, LH)], gsems[b])

    def drain_gather(r, b):
        pltpu.make_async_copy(table_hbm.at[idx_all.at[r, 0]],
                              rows_bufs[b].at[pl.ds(0, LH)], gsems[b]).wait()
        pltpu.make_async_copy(table_hbm.at[idx_all.at[r, 1]],
                              rows_bufs[b].at[pl.ds(LH, LH)], gsems[b]).wait()

    for b in range(NBUF):
        start_gather(b, b)

    def outer(g, carry):
        r0 = g * NBUF
        for b in range(NBUF):
            r = r0 + b
            drain_gather(r, b)
            out_cp = pltpu.make_async_copy(
                rows_bufs[b], embs_hbm.at[pl.ds((base + r) * L, L)], osems[b])
            out_cp.start()

            ln = lens_v[pl.ds(r, 16)][0]

            def acc_body(i, acc, _b=b):
                return tuple(acc[q] + rows_bufs[_b][i, pl.ds(q * 16, 16)]
                             for q in range(LG))

            acc = lax.fori_loop(
                0, ln, acc_body,
                tuple(jnp.zeros((16,), jnp.float32) for _ in range(LG)))
            lf_v = jnp.full((16,), ln.astype(jnp.float32))
            for q in range(LG):
                seq_acc[r, pl.ds(q * 16, 16)] = acc[q] / lf_v

            out_cp.wait()

            @pl.when(r + NBUF < BPW)
            def _():
                start_gather(r + NBUF, b)
        return carry

    lax.fori_loop(0, BPW // NBUF, outer, 0)
    pltpu.sync_copy(seq_acc, seq_hbm.at[pl.ds(base, BPW)])


@functools.partial(jax.jit, static_argnames=())
def _run(text_r, text_len, table):
    mesh = plsc.VectorSubcoreMesh(core_axis_name="c", subcore_axis_name="s")
    k = pl.kernel(
        _sc_body,
        mesh=mesh,
        out_type=[
            jax.ShapeDtypeStruct((B * L, D), jnp.float32),
            jax.ShapeDtypeStruct((B, D), jnp.float32),
        ],
        scratch_types=(
            [
                pltpu.VMEM((BPW, 2, LH), jnp.int32),
                pltpu.VMEM((BPW, D), jnp.float32),
                pltpu.VMEM((BPW + 16,), jnp.int32),
            ]
            + [pltpu.VMEM((L, D), jnp.float32) for _ in range(NBUF)]
            + [pltpu.SemaphoreType.DMA for _ in range(2 * NBUF)]
        ),
        compiler_params=pltpu.CompilerParams(use_tc_tiling_on_sc=False),
    )
    return k(text_r, text_len, table)


def kernel(text, text_len, table):
    text_r = text.astype(jnp.int32).reshape(B, 2, LH)
    embs_flat, seq = _run(text_r, text_len.astype(jnp.int32), table)
    return embs_flat.reshape(B, L, D), seq


# speedup vs baseline: 1.1175x; 1.0053x over previous
import functools

import jax
import jax.numpy as jnp
from jax import lax
from jax.experimental import pallas as pl
from jax.experimental.pallas import tpu as pltpu
from jax.experimental.pallas import tpu_sc as plsc

B = 4096
L = 200
D = 64
NW = 32
BPW = B // NW
LG = D // 16
LH = L // 2


NBUF = 4


def _sc_body(text_hbm, len_hbm, table_hbm, embs_hbm, seq_hbm,
             idx_all, seq_acc, lens_v, *bufs_and_sems):
    rows_bufs = bufs_and_sems[:NBUF]
    gsems = bufs_and_sems[NBUF:2 * NBUF]
    osems = bufs_and_sems[2 * NBUF:3 * NBUF]
    c = lax.axis_index("c")
    s = lax.axis_index("s")
    wid = s * 2 + c
    base = wid * BPW

    pltpu.sync_copy(text_hbm.at[pl.ds(base, BPW)], idx_all)
    pltpu.sync_copy(len_hbm.at[pl.ds(base, BPW)], lens_v.at[pl.ds(0, BPW)])

    def start_gather(r, b):
        pltpu.async_copy(table_hbm.at[idx_all.at[r, pl.ds(0, 128)]],
                         rows_bufs[b].at[pl.ds(0, 128)], gsems[b])
        pltpu.async_copy(table_hbm.at[idx_all.at[r, pl.ds(128, 72)]],
                         rows_bufs[b].at[pl.ds(128, 72)], gsems[b])

    def drain_gather(r, b):
        pltpu.make_async_copy(table_hbm.at[idx_all.at[r, pl.ds(0, 128)]],
                              rows_bufs[b].at[pl.ds(0, 128)], gsems[b]).wait()
        pltpu.make_async_copy(table_hbm.at[idx_all.at[r, pl.ds(128, 72)]],
                              rows_bufs[b].at[pl.ds(128, 72)], gsems[b]).wait()

    for b in range(NBUF):
        start_gather(b, b)

    def outer(g, carry):
        r0 = g * NBUF
        for b in range(NBUF):
            r = r0 + b
            drain_gather(r, b)
            out_cp = pltpu.make_async_copy(
                rows_bufs[b], embs_hbm.at[base + r], osems[b])
            out_cp.start()

            ln = lens_v[pl.ds(r, 16)][0]

            def acc_body(i, acc, _b=b):
                return tuple(acc[q] + rows_bufs[_b][i, pl.ds(q * 16, 16)]
                             for q in range(LG))

            acc = lax.fori_loop(
                0, ln, acc_body,
                tuple(jnp.zeros((16,), jnp.float32) for _ in range(LG)))
            lf_v = jnp.full((16,), ln.astype(jnp.float32))
            for q in range(LG):
                seq_acc[r, pl.ds(q * 16, 16)] = acc[q] / lf_v

            out_cp.wait()

            @pl.when(r + NBUF < BPW)
            def _():
                start_gather(r + NBUF, b)
        return carry

    lax.fori_loop(0, BPW // NBUF, outer, 0)
    pltpu.sync_copy(seq_acc, seq_hbm.at[pl.ds(base, BPW)])


@functools.partial(jax.jit, static_argnames=())
def _run(text_r, text_len, table):
    mesh = plsc.VectorSubcoreMesh(core_axis_name="c", subcore_axis_name="s")
    k = pl.kernel(
        _sc_body,
        mesh=mesh,
        out_type=[
            jax.ShapeDtypeStruct((B, L, D), jnp.float32),
            jax.ShapeDtypeStruct((B, D), jnp.float32),
        ],
        scratch_types=(
            [
                pltpu.VMEM((BPW, L), jnp.int32),
                pltpu.VMEM((BPW, D), jnp.float32),
                pltpu.VMEM((BPW + 16,), jnp.int32),
            ]
            + [pltpu.VMEM((L, D), jnp.float32) for _ in range(NBUF)]
            + [pltpu.SemaphoreType.DMA for _ in range(2 * NBUF)]
        ),
        compiler_params=pltpu.CompilerParams(use_tc_tiling_on_sc=False),
    )
    return k(text_r, text_len, table)


def kernel(text, text_len, table):
    embs, seq = _run(text.astype(jnp.int32), text_len.astype(jnp.int32), table)
    return embs, seq
